# Initial kernel scaffold; baseline (speedup 1.0000x reference)
import jax
import jax.numpy as jnp
from jax.experimental import pallas as pl


def kernel(x_nchw, w3_hwio, w1, gamma1, beta1, gamma2, beta2):
    raise NotImplementedError("write your pallas kernel here")


# speedup vs baseline: 4.4975x; 4.4975x over previous
import functools

import jax
import jax.numpy as jnp
from jax.experimental import pallas as pl
from jax.experimental.pallas import tpu as pltpu

_EPS = 1e-5


def _conv3_kernel(x_ref, w_ref, y_ref, st_ref, *, B, H, W):
    HW = H * W
    LB = B * HW
    xb = jnp.concatenate([x_ref[b] for b in range(B)],
                         axis=1).astype(jnp.bfloat16)
    lane = jax.lax.broadcasted_iota(jnp.int32, (1, LB), 1)
    wpos = lane % W
    hpos = (lane // W) % H
    taps = []
    for dy in range(3):
        for dx in range(3):
            off = (dy - 1) * W + (dx - 1)
            if off == 0:
                sheet = xb
            else:
                r = off % LB
                sheet = jnp.concatenate([xb[:, r:], xb[:, :r]], axis=1)
            conds = []
            if dy == 0:
                conds.append(hpos >= 1)
            if dy == 2:
                conds.append(hpos <= H - 2)
            if dx == 0:
                conds.append(wpos >= 1)
            if dx == 2:
                conds.append(wpos <= W - 2)
            if conds:
                m = conds[0]
                for c in conds[1:]:
                    m = jnp.logical_and(m, c)
                sheet = jnp.where(m, sheet, jnp.bfloat16(0))
            taps.append(sheet)
    a9 = jnp.concatenate(taps, axis=0)
    y = jax.lax.dot_general(w_ref[...], a9, (((1,), (0,)), ((), ())),
                            preferred_element_type=jnp.float32)
    s = jnp.sum(y, axis=1, keepdims=True)
    ss = jnp.sum(y * y, axis=1, keepdims=True)
    st_ref[0] = jnp.concatenate([s, ss], axis=1)
    y_ref[0] = y.astype(jnp.bfloat16)


def _bn_conv1_kernel(y_ref, sc_ref, sh_ref, w_ref, z_ref, st_ref):
    a = jnp.maximum(y_ref[0].astype(jnp.float32) * sc_ref[...] + sh_ref[...],
                    0.0)
    z = jax.lax.dot_general(w_ref[...], a.astype(jnp.bfloat16),
                            (((1,), (0,)), ((), ())),
                            preferred_element_type=jnp.float32)
    s = jnp.sum(z, axis=1, keepdims=True)
    ss = jnp.sum(z * z, axis=1, keepdims=True)
    st_ref[0] = jnp.concatenate([s, ss], axis=1)
    z_ref[0] = z.astype(jnp.bfloat16)


def _bn_out_kernel(z_ref, sc_ref, sh_ref, o_ref, *, B, HW):
    o = jnp.maximum(z_ref[0].astype(jnp.float32) * sc_ref[...] + sh_ref[...],
                    0.0)
    for b in range(B):
        o_ref[b] = o[:, b * HW:(b + 1) * HW]


def _fold_bn(st, gamma, beta, count):
    tot = jnp.sum(st.astype(jnp.float32), axis=0)
    mean = tot[:, 0] / count
    var = tot[:, 1] / count - mean * mean
    scale = gamma * jax.lax.rsqrt(var + _EPS)
    shift = beta - mean * scale
    return scale.reshape(-1, 1), shift.reshape(-1, 1)


@jax.jit
def _forward(x_nchw, w3_hwio, w1, gamma1, beta1, gamma2, beta2):
    N, Cin, H, W = x_nchw.shape
    Cout = w3_hwio.shape[-1]
    HW = H * W
    B = 16 if N % 16 == 0 else (8 if N % 8 == 0 else 1)
    S = N // B
    LB = B * HW
    parallel = pltpu.CompilerParams(dimension_semantics=("parallel",))

    x3 = x_nchw.reshape(N, Cin, HW)
    w9t = jnp.transpose(w3_hwio, (3, 0, 1, 2)).reshape(
        Cout, 9 * Cin).astype(jnp.bfloat16)
    w1t = jnp.transpose(w1).astype(jnp.bfloat16)

    y1, st1 = pl.pallas_call(
        functools.partial(_conv3_kernel, B=B, H=H, W=W),
        grid=(S,),
        in_specs=[
            pl.BlockSpec((B, Cin, HW), lambda i: (i, 0, 0)),
            pl.BlockSpec((Cout, 9 * Cin), lambda i: (0, 0)),
        ],
        out_specs=[
            pl.BlockSpec((1, Cout, LB), lambda i: (i, 0, 0)),
            pl.BlockSpec((1, Cout, 2), lambda i: (i, 0, 0)),
        ],
        out_shape=[
            jax.ShapeDtypeStruct((S, Cout, LB), jnp.bfloat16),
            jax.ShapeDtypeStruct((S, Cout, 2), jnp.float32),
        ],
        compiler_params=parallel,
    )(x3, w9t)

    sc1, sh1 = _fold_bn(st1, gamma1, beta1, N * HW)

    z, st2 = pl.pallas_call(
        _bn_conv1_kernel,
        grid=(S,),
        in_specs=[
            pl.BlockSpec((1, Cout, LB), lambda i: (i, 0, 0)),
            pl.BlockSpec((Cout, 1), lambda i: (0, 0)),
            pl.BlockSpec((Cout, 1), lambda i: (0, 0)),
            pl.BlockSpec((Cout, Cout), lambda i: (0, 0)),
        ],
        out_specs=[
            pl.BlockSpec((1, Cout, LB), lambda i: (i, 0, 0)),
            pl.BlockSpec((1, Cout, 2), lambda i: (i, 0, 0)),
        ],
        out_shape=[
            jax.ShapeDtypeStruct((S, Cout, LB), jnp.bfloat16),
            jax.ShapeDtypeStruct((S, Cout, 2), jnp.float32),
        ],
        compiler_params=parallel,
    )(y1, sc1, sh1, w1t)

    sc2, sh2 = _fold_bn(st2, gamma2, beta2, N * HW)

    out3 = pl.pallas_call(
        functools.partial(_bn_out_kernel, B=B, HW=HW),
        grid=(S,),
        in_specs=[
            pl.BlockSpec((1, Cout, LB), lambda i: (i, 0, 0)),
            pl.BlockSpec((Cout, 1), lambda i: (0, 0)),
            pl.BlockSpec((Cout, 1), lambda i: (0, 0)),
        ],
        out_specs=pl.BlockSpec((B, Cout, HW), lambda i: (i, 0, 0)),
        out_shape=jax.ShapeDtypeStruct((N, Cout, HW), jnp.float32),
        compiler_params=parallel,
    )(z, sc2, sh2)

    return out3.reshape(N, Cout, H, W)


def kernel(x_nchw, w3_hwio, w1, gamma1, beta1, gamma2, beta2):
    return _forward(x_nchw, w3_hwio, w1, gamma1, beta1, gamma2, beta2)


# speedup vs baseline: 5.0106x; 1.1141x over previous
import functools

import jax
import jax.numpy as jnp
from jax.experimental import pallas as pl
from jax.experimental.pallas import tpu as pltpu

_EPS = 1e-5


def _conv3_kernel(x_ref, w_ref, y_ref, st_ref, *, B, H, W):
    HW = H * W
    LB = B * HW
    xb = jnp.concatenate([x_ref[b] for b in range(B)],
                         axis=1).astype(jnp.bfloat16)
    lane = jax.lax.broadcasted_iota(jnp.int32, (1, LB), 1)
    wpos = lane % W
    hpos = (lane // W) % H
    taps = []
    for dy in range(3):
        for dx in range(3):
            off = (dy - 1) * W + (dx - 1)
            if off == 0:
                sheet = xb
            else:
                r = off % LB
                sheet = jnp.concatenate([xb[:, r:], xb[:, :r]], axis=1)
            conds = []
            if dy == 0:
                conds.append(hpos >= 1)
            if dy == 2:
                conds.append(hpos <= H - 2)
            if dx == 0:
                conds.append(wpos >= 1)
            if dx == 2:
                conds.append(wpos <= W - 2)
            if conds:
                m = conds[0]
                for c in conds[1:]:
                    m = jnp.logical_and(m, c)
                sheet = jnp.where(m, sheet, jnp.bfloat16(0))
            taps.append(sheet)
    a9 = jnp.concatenate(taps, axis=0)
    y = jax.lax.dot_general(w_ref[...], a9, (((1,), (0,)), ((), ())),
                            preferred_element_type=jnp.float32)
    s = jnp.sum(y, axis=1, keepdims=True)
    ss = jnp.sum(y * y, axis=1, keepdims=True)
    st_ref[0] = jnp.concatenate([s, ss], axis=1)
    y_ref[0] = y.astype(jnp.bfloat16)


def _bn_conv1_kernel(y_ref, sc_ref, sh_ref, w_ref, z_ref, st_ref):
    a = jnp.maximum(y_ref[0].astype(jnp.float32) * sc_ref[...] + sh_ref[...],
                    0.0)
    z = jax.lax.dot_general(w_ref[...], a.astype(jnp.bfloat16),
                            (((1,), (0,)), ((), ())),
                            preferred_element_type=jnp.float32)
    s = jnp.sum(z, axis=1, keepdims=True)
    ss = jnp.sum(z * z, axis=1, keepdims=True)
    st_ref[0] = jnp.concatenate([s, ss], axis=1)
    z_ref[0] = z.astype(jnp.bfloat16)


def _bn_out_kernel(z_ref, sc_ref, sh_ref, o_ref, *, B, HW):
    o = jnp.maximum(z_ref[0].astype(jnp.float32) * sc_ref[...] + sh_ref[...],
                    0.0)
    for b in range(B):
        o_ref[b] = o[:, b * HW:(b + 1) * HW]


def _fold_bn(st, gamma, beta, count):
    tot = jnp.sum(st.astype(jnp.float32), axis=0)
    mean = tot[:, 0] / count
    var = tot[:, 1] / count - mean * mean
    scale = gamma * jax.lax.rsqrt(var + _EPS)
    shift = beta - mean * scale
    return scale.reshape(-1, 1), shift.reshape(-1, 1)


@jax.jit
def _forward(x_nchw, w3_hwio, w1, gamma1, beta1, gamma2, beta2):
    N, Cin, H, W = x_nchw.shape
    Cout = w3_hwio.shape[-1]
    HW = H * W
    B = 32 if N % 32 == 0 else (8 if N % 8 == 0 else 1)
    S = N // B
    LB = B * HW
    parallel = pltpu.CompilerParams(dimension_semantics=("parallel",))

    x3 = x_nchw.reshape(N, Cin, HW)
    w9t = jnp.transpose(w3_hwio, (3, 0, 1, 2)).reshape(
        Cout, 9 * Cin).astype(jnp.bfloat16)
    w1t = jnp.transpose(w1).astype(jnp.bfloat16)

    y1, st1 = pl.pallas_call(
        functools.partial(_conv3_kernel, B=B, H=H, W=W),
        grid=(S,),
        in_specs=[
            pl.BlockSpec((B, Cin, HW), lambda i: (i, 0, 0)),
            pl.BlockSpec((Cout, 9 * Cin), lambda i: (0, 0)),
        ],
        out_specs=[
            pl.BlockSpec((1, Cout, LB), lambda i: (i, 0, 0)),
            pl.BlockSpec((1, Cout, 2), lambda i: (i, 0, 0)),
        ],
        out_shape=[
            jax.ShapeDtypeStruct((S, Cout, LB), jnp.bfloat16),
            jax.ShapeDtypeStruct((S, Cout, 2), jnp.float32),
        ],
        compiler_params=parallel,
    )(x3, w9t)

    sc1, sh1 = _fold_bn(st1, gamma1, beta1, N * HW)

    z, st2 = pl.pallas_call(
        _bn_conv1_kernel,
        grid=(S,),
        in_specs=[
            pl.BlockSpec((1, Cout, LB), lambda i: (i, 0, 0)),
            pl.BlockSpec((Cout, 1), lambda i: (0, 0)),
            pl.BlockSpec((Cout, 1), lambda i: (0, 0)),
            pl.BlockSpec((Cout, Cout), lambda i: (0, 0)),
        ],
        out_specs=[
            pl.BlockSpec((1, Cout, LB), lambda i: (i, 0, 0)),
            pl.BlockSpec((1, Cout, 2), lambda i: (i, 0, 0)),
        ],
        out_shape=[
            jax.ShapeDtypeStruct((S, Cout, LB), jnp.bfloat16),
            jax.ShapeDtypeStruct((S, Cout, 2), jnp.float32),
        ],
        compiler_params=parallel,
    )(y1, sc1, sh1, w1t)

    sc2, sh2 = _fold_bn(st2, gamma2, beta2, N * HW)

    out3 = pl.pallas_call(
        functools.partial(_bn_out_kernel, B=B, HW=HW),
        grid=(S,),
        in_specs=[
            pl.BlockSpec((1, Cout, LB), lambda i: (i, 0, 0)),
            pl.BlockSpec((Cout, 1), lambda i: (0, 0)),
            pl.BlockSpec((Cout, 1), lambda i: (0, 0)),
        ],
        out_specs=pl.BlockSpec((B, Cout, HW), lambda i: (i, 0, 0)),
        out_shape=jax.ShapeDtypeStruct((N, Cout, HW), jnp.float32),
        compiler_params=parallel,
    )(z, sc2, sh2)

    return out3.reshape(N, Cout, H, W)


def kernel(x_nchw, w3_hwio, w1, gamma1, beta1, gamma2, beta2):
    return _forward(x_nchw, w3_hwio, w1, gamma1, beta1, gamma2, beta2)
